# Initial kernel scaffold; baseline (speedup 1.0000x reference)
import jax
import jax.numpy as jnp
from jax.experimental import pallas as pl


def kernel(x, edge_index_et0, edge_index_et1, W1_et0, W2_et0, W1_et1, W2_et1):
    raise NotImplementedError("write your pallas kernel here")


# speedup vs baseline: 1.0546x; 1.0546x over previous
import functools

import jax
import jax.numpy as jnp
from jax import lax
from jax.experimental import pallas as pl
from jax.experimental.pallas import tpu as pltpu
from jax.experimental.pallas import tpu_sc as plsc

N_NODES = 10000
DIM = 128
N_PAD = 10240
BM = 512
NC, NS = 2, 16
NW = NC * NS
CH = 128
E_PAD = 163840
EPW = E_PAD // NW
NCH = EPW // CH
RPS = N_PAD // NS
CW = 16

_f32 = jnp.float32


def _mlp_block(x_ref, w1_ref, w2_ref, y_ref):
    h = jnp.maximum(
        jnp.dot(x_ref[...], w1_ref[...], preferred_element_type=_f32), 0.0)
    y_ref[...] = jnp.dot(h, w2_ref[...], preferred_element_type=_f32)


def _combine_mlp_block(x_ref, sums_ref, cnt_ref, w1_ref, w2_ref,
                       data_ref, y_ref):
    cnt = cnt_ref[0] + cnt_ref[1]
    sums = sums_ref[0] + sums_ref[1]
    mean = sums / jnp.maximum(cnt, 1.0)
    data = jnp.where(cnt > 0, mean, x_ref[...])
    data_ref[...] = data
    h = jnp.maximum(jnp.dot(data, w1_ref[...], preferred_element_type=_f32),
                    0.0)
    y_ref[...] = jnp.dot(h, w2_ref[...], preferred_element_type=_f32)


def _combine_block(x_ref, sums_ref, cnt_ref, data_ref):
    cnt = cnt_ref[0] + cnt_ref[1]
    sums = sums_ref[0] + sums_ref[1]
    mean = sums / jnp.maximum(cnt, 1.0)
    data_ref[...] = jnp.where(cnt > 0, mean, x_ref[...])


_x_spec = pl.BlockSpec((BM, DIM), lambda i: (i, 0))
_w1_spec = pl.BlockSpec((DIM, 256), lambda i: (0, 0))
_w2_spec = pl.BlockSpec((256, DIM), lambda i: (0, 0))
_sums_spec = pl.BlockSpec((NC, BM, DIM), lambda i: (0, i, 0))
_cnt_spec = pl.BlockSpec((NC, BM, 1), lambda i: (0, i, 0))
_GRID = (N_PAD // BM,)


def _mlp(x, w1, w2):
    return pl.pallas_call(
        _mlp_block,
        grid=_GRID,
        in_specs=[_x_spec, _w1_spec, _w2_spec],
        out_specs=_x_spec,
        out_shape=jax.ShapeDtypeStruct((N_PAD, DIM), _f32),
    )(x, w1, w2)


def _combine_mlp(x, sums_p, cnt_p, w1, w2):
    return pl.pallas_call(
        _combine_mlp_block,
        grid=_GRID,
        in_specs=[_x_spec, _sums_spec, _cnt_spec, _w1_spec, _w2_spec],
        out_specs=[_x_spec, _x_spec],
        out_shape=[jax.ShapeDtypeStruct((N_PAD, DIM), _f32),
                   jax.ShapeDtypeStruct((N_PAD, DIM), _f32)],
    )(x, sums_p, cnt_p, w1, w2)


def _combine(x, sums_p, cnt_p):
    return pl.pallas_call(
        _combine_block,
        grid=_GRID,
        in_specs=[_x_spec, _sums_spec, _cnt_spec],
        out_specs=_x_spec,
        out_shape=jax.ShapeDtypeStruct((N_PAD, DIM), _f32),
    )(x, sums_p, cnt_p)


CNT_R = N_PAD // 8


def _sc_body(y_hbm, src_hbm, dst_hbm, div_hbm, mod_hbm, nid_hbm, zb_hbm,
             oh_hbm, sums_out, cnt_out,
             acc_sh, cnt_sh, src_idx, dst_idx, div_idx, mod_idx, rows, sem):
    c = lax.axis_index("c")
    s = lax.axis_index("s")
    pltpu.sync_copy(zb_hbm, rows)

    @pl.loop(0, RPS // CH)
    def _init(j):
        off = s * RPS + j * CH
        pltpu.sync_copy(nid_hbm.at[pl.ds(off, CH)], dst_idx)
        pltpu.sync_copy(rows, acc_sh.at[dst_idx])

    @pl.loop(0, CNT_R // CH)
    def _initc(j):
        pltpu.sync_copy(nid_hbm.at[pl.ds(j * CH, CH)], dst_idx)
        pltpu.sync_copy(rows, cnt_sh.at[dst_idx])

    plsc.subcore_barrier()

    base = (c * NS + s) * EPW

    @pl.loop(0, NCH)
    def _step(i):
        off = base + i * CH
        pltpu.sync_copy(src_hbm.at[pl.ds(off, CH)], src_idx)
        pltpu.sync_copy(dst_hbm.at[pl.ds(off, CH)], dst_idx)
        pltpu.sync_copy(div_hbm.at[pl.ds(off, CH)], div_idx)
        pltpu.sync_copy(mod_hbm.at[pl.ds(off, CH)], mod_idx)
        pltpu.async_copy(y_hbm.at[src_idx], rows, sem).wait()
        pltpu.sync_copy(rows, acc_sh.at[dst_idx], add=True)
        pltpu.async_copy(oh_hbm.at[mod_idx], rows, sem).wait()
        pltpu.sync_copy(rows, cnt_sh.at[div_idx], add=True)

    plsc.subcore_barrier()

    @pl.loop(0, RPS // CH)
    def _out(j):
        off = s * RPS + j * CH
        pltpu.sync_copy(nid_hbm.at[pl.ds(off, CH)], dst_idx)
        pltpu.async_copy(acc_sh.at[dst_idx], rows, sem).wait()
        pltpu.sync_copy(rows, sums_out.at[pl.ds(c * N_PAD + off, CH)])

    @pl.loop(0, CNT_R // CH)
    def _outc(j):
        pltpu.sync_copy(nid_hbm.at[pl.ds(j * CH, CH)], dst_idx)
        pltpu.async_copy(cnt_sh.at[dst_idx], rows, sem).wait()
        pltpu.sync_copy(rows, cnt_out.at[pl.ds(c * CNT_R + j * CH, CH)])


def _sc_scatter(y, src, dst, dst_div, dst_mod, nid, zb, onehot):
    mesh = plsc.VectorSubcoreMesh(core_axis_name="c", subcore_axis_name="s")
    fn = pl.kernel(
        _sc_body,
        out_type=[jax.ShapeDtypeStruct((NC * N_PAD, DIM), _f32),
                  jax.ShapeDtypeStruct((NC * CNT_R, DIM), _f32)],
        mesh=mesh,
        scratch_types=[
            pltpu.VMEM_SHARED((N_PAD, DIM), _f32),
            pltpu.VMEM_SHARED((CNT_R, DIM), _f32),
            pltpu.VMEM((CH,), jnp.int32),
            pltpu.VMEM((CH,), jnp.int32),
            pltpu.VMEM((CH,), jnp.int32),
            pltpu.VMEM((CH,), jnp.int32),
            pltpu.VMEM((CH, DIM), _f32),
            pltpu.SemaphoreType.DMA,
        ],
    )
    sums_flat, cnt_flat = fn(y, src, dst, dst_div, dst_mod, nid, zb, onehot)
    counts = cnt_flat.reshape(NC, CNT_R, DIM)[:, :, :8].reshape(NC, N_PAD, 1)
    return sums_flat.reshape(NC, N_PAD, DIM), counts


def kernel(x, edge_index_et0, edge_index_et1, W1_et0, W2_et0, W1_et1, W2_et1):
    x_pad = jnp.pad(x, ((0, N_PAD - N_NODES), (0, 0)))
    n_extra = E_PAD - edge_index_et0.shape[1]
    pad_rows = (N_NODES + jnp.arange(n_extra, dtype=jnp.int32)
                % (N_PAD - N_NODES))
    def _split(ei):
        src = jnp.concatenate([ei[0].astype(jnp.int32), pad_rows])
        dst = jnp.concatenate([ei[1].astype(jnp.int32), pad_rows])
        return src, dst, dst >> 3, dst & 7
    src0, dst0, div0, mod0 = _split(edge_index_et0)
    src1, dst1, div1, mod1 = _split(edge_index_et1)

    nid = jnp.arange(N_PAD, dtype=jnp.int32)
    zb = jnp.zeros((CH, DIM), _f32)
    onehot = (jnp.arange(8)[:, None] == jnp.arange(DIM)[None, :]).astype(_f32)

    y0 = _mlp(x_pad, W1_et0, W2_et0)
    sums0, cnt0 = _sc_scatter(y0, src0, dst0, div0, mod0, nid, zb, onehot)
    data1, y1 = _combine_mlp(x_pad, sums0, cnt0, W1_et1, W2_et1)
    sums1, cnt1 = _sc_scatter(y1, src1, dst1, div1, mod1, nid, zb, onehot)
    data2 = _combine(data1, sums1, cnt1)
    return data2[:N_NODES]
